# Initial kernel scaffold; baseline (speedup 1.0000x reference)
import jax
import jax.numpy as jnp
from jax.experimental import pallas as pl


def kernel(hidden_states, position_index, behavior_index, behavior_table, W_gate, W_up, W_down):
    raise NotImplementedError("write your pallas kernel here")


# speedup vs baseline: 2.4730x; 2.4730x over previous
import functools

import jax
import jax.numpy as jnp
from jax import lax
from jax.experimental import pallas as pl
from jax.experimental.pallas import tpu as pltpu
from jax.experimental.pallas import tpu_sc as plsc

NUM_EXPERTS = 8
D_MODEL = 1024
D_BEH = 64
D_FF = 2048
TOKENS = 8192

TILE = 256
NUM_TILES = TOKENS // TILE + NUM_EXPERTS
PAD = NUM_TILES * TILE

NUM_WORKERS = 32
CHUNK = 64


def _route(position_index):
    pi = position_index.astype(jnp.int32)
    onehot = (pi[:, None] == jnp.arange(NUM_EXPERTS, dtype=jnp.int32)[None, :])
    ranks_all = jnp.cumsum(onehot.astype(jnp.int32), axis=0)
    counts = ranks_all[-1]
    rank = jnp.take_along_axis(ranks_all, pi[:, None], axis=1)[:, 0] - 1
    tiles_per_e = (counts + TILE - 1) // TILE
    pad_start = (jnp.concatenate([jnp.zeros((1,), jnp.int32),
                                  jnp.cumsum(tiles_per_e)])[:NUM_EXPERTS]
                 * TILE).astype(jnp.int32)
    dest = pad_start[pi] + rank
    src_idx = jnp.zeros((PAD,), jnp.int32).at[dest].set(
        jnp.arange(TOKENS, dtype=jnp.int32))
    tile_starts = jnp.arange(NUM_TILES, dtype=jnp.int32) * TILE
    tile_expert = (jnp.searchsorted(pad_start, tile_starts, side="right")
                   .astype(jnp.int32) - 1)
    tile_expert = jnp.clip(tile_expert, 0, NUM_EXPERTS - 1)
    return dest, src_idx, tile_expert


def _sc_mesh():
    return plsc.VectorSubcoreMesh(core_axis_name="c", subcore_axis_name="s")


def _dispatch_gather(hidden_states, behavior_table_pad, src_idx, bidx_sorted):
    rows_per_w = PAD // NUM_WORKERS
    n_chunks = rows_per_w // CHUNK

    @functools.partial(
        pl.kernel,
        out_type=[jax.ShapeDtypeStruct((PAD, D_MODEL), jnp.float32),
                  jax.ShapeDtypeStruct((PAD, 128), jnp.float32)],
        mesh=_sc_mesh(),
        scratch_types=[pltpu.VMEM((CHUNK,), jnp.int32),
                       pltpu.VMEM((CHUNK,), jnp.int32),
                       pltpu.VMEM((CHUNK, D_MODEL), jnp.float32),
                       pltpu.VMEM((CHUNK, 128), jnp.float32),
                       pltpu.SemaphoreType.DMA,
                       pltpu.SemaphoreType.DMA],
    )
    def k(hid_hbm, beh_hbm, src_hbm, bidx_hbm, out_h, out_b,
          idx_v, bidx_v, rows_v, brows_v, sem_h, sem_b):
        wid = lax.axis_index("s") * 2 + lax.axis_index("c")
        base = wid * rows_per_w
        for c in range(n_chunks):
            off = base + c * CHUNK
            pltpu.sync_copy(src_hbm.at[pl.ds(off, CHUNK)], idx_v)
            pltpu.sync_copy(bidx_hbm.at[pl.ds(off, CHUNK)], bidx_v)
            cp_h = pltpu.async_copy(hid_hbm.at[idx_v], rows_v, sem_h)
            cp_b = pltpu.async_copy(beh_hbm.at[bidx_v], brows_v, sem_b)
            cp_h.wait()
            cp_b.wait()
            pltpu.sync_copy(rows_v, out_h.at[pl.ds(off, CHUNK)])
            pltpu.sync_copy(brows_v, out_b.at[pl.ds(off, CHUNK)])

    return k(hidden_states, behavior_table_pad, src_idx, bidx_sorted)


def _combine_gather(out_sorted, dest):
    rows_per_w = TOKENS // NUM_WORKERS
    n_chunks = rows_per_w // CHUNK

    @functools.partial(
        pl.kernel,
        out_type=jax.ShapeDtypeStruct((TOKENS, D_MODEL), jnp.float32),
        mesh=_sc_mesh(),
        scratch_types=[pltpu.VMEM((CHUNK,), jnp.int32),
                       pltpu.VMEM((CHUNK, D_MODEL), jnp.float32),
                       pltpu.SemaphoreType.DMA],
    )
    def k(osort_hbm, dest_hbm, out_hbm, idx_v, rows_v, sem):
        wid = lax.axis_index("s") * 2 + lax.axis_index("c")
        base = wid * rows_per_w
        for c in range(n_chunks):
            off = base + c * CHUNK
            pltpu.sync_copy(dest_hbm.at[pl.ds(off, CHUNK)], idx_v)
            pltpu.async_copy(osort_hbm.at[idx_v], rows_v, sem).wait()
            pltpu.sync_copy(rows_v, out_hbm.at[pl.ds(off, CHUNK)])

    return k(out_sorted, dest)


def _mlp_body(te_ref, xh_ref, xb_ref, wgh_ref, wgb_ref, wuh_ref, wub_ref,
              wd_ref, out_ref):
    xh = xh_ref[...]
    xb = xb_ref[:, :D_BEH]
    g = (jnp.dot(xh, wgh_ref[0], preferred_element_type=jnp.float32)
         + jnp.dot(xb, wgb_ref[0], preferred_element_type=jnp.float32))
    u = (jnp.dot(xh, wuh_ref[0], preferred_element_type=jnp.float32)
         + jnp.dot(xb, wub_ref[0], preferred_element_type=jnp.float32))
    a = (g * lax.logistic(g)) * u
    out_ref[...] = jnp.dot(a, wd_ref[0], preferred_element_type=jnp.float32)


def _grouped_mlp(xh, xb, W_gate, W_up, W_down, tile_expert):
    boff = D_MODEL // D_BEH
    grid_spec = pltpu.PrefetchScalarGridSpec(
        num_scalar_prefetch=1,
        grid=(NUM_TILES,),
        in_specs=[
            pl.BlockSpec((TILE, D_MODEL), lambda i, te: (i, 0)),
            pl.BlockSpec((TILE, 128), lambda i, te: (i, 0)),
            pl.BlockSpec((1, D_MODEL, D_FF), lambda i, te: (te[i], 0, 0)),
            pl.BlockSpec((1, D_BEH, D_FF), lambda i, te: (te[i], boff, 0)),
            pl.BlockSpec((1, D_MODEL, D_FF), lambda i, te: (te[i], 0, 0)),
            pl.BlockSpec((1, D_BEH, D_FF), lambda i, te: (te[i], boff, 0)),
            pl.BlockSpec((1, D_FF, D_MODEL), lambda i, te: (te[i], 0, 0)),
        ],
        out_specs=pl.BlockSpec((TILE, D_MODEL), lambda i, te: (i, 0)),
    )
    return pl.pallas_call(
        _mlp_body,
        grid_spec=grid_spec,
        out_shape=jax.ShapeDtypeStruct((PAD, D_MODEL), jnp.float32),
        compiler_params=pltpu.CompilerParams(
            vmem_limit_bytes=100 * 1024 * 1024),
    )(tile_expert, xh, xb, W_gate, W_gate, W_up, W_up, W_down)


@jax.jit
def kernel(hidden_states, position_index, behavior_index, behavior_table,
           W_gate, W_up, W_down):
    dest, src_idx, tile_expert = _route(position_index)
    bidx_sorted = behavior_index.astype(jnp.int32)[src_idx]
    bt_pad = jnp.zeros((behavior_table.shape[0], 128), jnp.float32)
    bt_pad = bt_pad.at[:, :D_BEH].set(behavior_table)
    xh, xb = _dispatch_gather(hidden_states, bt_pad, src_idx, bidx_sorted)
    out_sorted = _grouped_mlp(xh, xb, W_gate, W_up, W_down, tile_expert)
    return _combine_gather(out_sorted, dest)


# speedup vs baseline: 2.4866x; 1.0055x over previous
import functools

import jax
import jax.numpy as jnp
from jax import lax
from jax.experimental import pallas as pl
from jax.experimental.pallas import tpu as pltpu
from jax.experimental.pallas import tpu_sc as plsc

NUM_EXPERTS = 8
D_MODEL = 1024
D_BEH = 64
D_FF = 2048
TOKENS = 8192

TILE = 256
NUM_TILES = TOKENS // TILE + NUM_EXPERTS
PAD = NUM_TILES * TILE

NUM_WORKERS = 32
CHUNK = 32


def _route(position_index):
    pi = position_index.astype(jnp.int32)
    onehot = (pi[:, None] == jnp.arange(NUM_EXPERTS, dtype=jnp.int32)[None, :])
    ranks_all = jnp.cumsum(onehot.astype(jnp.int32), axis=0)
    counts = ranks_all[-1]
    rank = jnp.take_along_axis(ranks_all, pi[:, None], axis=1)[:, 0] - 1
    tiles_per_e = (counts + TILE - 1) // TILE
    pad_start = (jnp.concatenate([jnp.zeros((1,), jnp.int32),
                                  jnp.cumsum(tiles_per_e)])[:NUM_EXPERTS]
                 * TILE).astype(jnp.int32)
    dest = pad_start[pi] + rank
    src_idx = jnp.zeros((PAD,), jnp.int32).at[dest].set(
        jnp.arange(TOKENS, dtype=jnp.int32))
    tile_starts = jnp.arange(NUM_TILES, dtype=jnp.int32) * TILE
    tile_expert = (jnp.searchsorted(pad_start, tile_starts, side="right")
                   .astype(jnp.int32) - 1)
    tile_expert = jnp.clip(tile_expert, 0, NUM_EXPERTS - 1)
    return dest, src_idx, tile_expert


def _sc_mesh():
    return plsc.VectorSubcoreMesh(core_axis_name="c", subcore_axis_name="s")


def _dispatch_gather(hidden_states, behavior_table_pad, src_idx, bidx_sorted):
    rows_per_w = PAD // NUM_WORKERS
    n_chunks = rows_per_w // CHUNK

    @functools.partial(
        pl.kernel,
        out_type=[jax.ShapeDtypeStruct((PAD, D_MODEL), jnp.float32),
                  jax.ShapeDtypeStruct((PAD, 128), jnp.float32)],
        mesh=_sc_mesh(),
        scratch_types=[pltpu.VMEM((2, CHUNK), jnp.int32),
                       pltpu.VMEM((2, CHUNK), jnp.int32),
                       pltpu.VMEM((2, CHUNK, D_MODEL), jnp.float32),
                       pltpu.VMEM((2, CHUNK, 128), jnp.float32),
                       pltpu.SemaphoreType.DMA,
                       pltpu.SemaphoreType.DMA,
                       pltpu.SemaphoreType.DMA,
                       pltpu.SemaphoreType.DMA],
    )
    def k(hid_hbm, beh_hbm, src_hbm, bidx_hbm, out_h, out_b,
          idx_v, bidx_v, rows_v, brows_v, sh0, sh1, sb0, sb1):
        wid = lax.axis_index("s") * 2 + lax.axis_index("c")
        base = wid * rows_per_w
        sems_h = (sh0, sh1)
        sems_b = (sb0, sb1)

        def start(c):
            b = c % 2
            off = base + c * CHUNK
            pltpu.sync_copy(src_hbm.at[pl.ds(off, CHUNK)], idx_v.at[b])
            pltpu.sync_copy(bidx_hbm.at[pl.ds(off, CHUNK)], bidx_v.at[b])
            cp_h = pltpu.async_copy(hid_hbm.at[idx_v.at[b]], rows_v.at[b],
                                    sems_h[b])
            cp_b = pltpu.async_copy(beh_hbm.at[bidx_v.at[b]], brows_v.at[b],
                                    sems_b[b])
            return cp_h, cp_b

        pend = {0: start(0)}
        for c in range(n_chunks):
            if c + 1 < n_chunks:
                pend[c + 1] = start(c + 1)
            cp_h, cp_b = pend.pop(c)
            cp_h.wait()
            cp_b.wait()
            b = c % 2
            off = base + c * CHUNK
            pltpu.sync_copy(rows_v.at[b], out_h.at[pl.ds(off, CHUNK)])
            pltpu.sync_copy(brows_v.at[b], out_b.at[pl.ds(off, CHUNK)])

    return k(hidden_states, behavior_table_pad, src_idx, bidx_sorted)


def _combine_gather(out_sorted, dest):
    rows_per_w = TOKENS // NUM_WORKERS
    n_chunks = rows_per_w // CHUNK

    @functools.partial(
        pl.kernel,
        out_type=jax.ShapeDtypeStruct((TOKENS, D_MODEL), jnp.float32),
        mesh=_sc_mesh(),
        scratch_types=[pltpu.VMEM((2, CHUNK), jnp.int32),
                       pltpu.VMEM((2, CHUNK, D_MODEL), jnp.float32),
                       pltpu.SemaphoreType.DMA,
                       pltpu.SemaphoreType.DMA],
    )
    def k(osort_hbm, dest_hbm, out_hbm, idx_v, rows_v, s0, s1):
        wid = lax.axis_index("s") * 2 + lax.axis_index("c")
        base = wid * rows_per_w
        sems = (s0, s1)

        def start(c):
            b = c % 2
            off = base + c * CHUNK
            pltpu.sync_copy(dest_hbm.at[pl.ds(off, CHUNK)], idx_v.at[b])
            return pltpu.async_copy(osort_hbm.at[idx_v.at[b]], rows_v.at[b],
                                    sems[b])

        pend = {0: start(0)}
        for c in range(n_chunks):
            if c + 1 < n_chunks:
                pend[c + 1] = start(c + 1)
            pend.pop(c).wait()
            b = c % 2
            off = base + c * CHUNK
            pltpu.sync_copy(rows_v.at[b], out_hbm.at[pl.ds(off, CHUNK)])

    return k(out_sorted, dest)


def _mlp_body(te_ref, xh_ref, xb_ref, wgh_ref, wgb_ref, wuh_ref, wub_ref,
              wd_ref, out_ref):
    xh = xh_ref[...]
    xb = xb_ref[:, :D_BEH]
    g = (jnp.dot(xh, wgh_ref[0], preferred_element_type=jnp.float32)
         + jnp.dot(xb, wgb_ref[0], preferred_element_type=jnp.float32))
    u = (jnp.dot(xh, wuh_ref[0], preferred_element_type=jnp.float32)
         + jnp.dot(xb, wub_ref[0], preferred_element_type=jnp.float32))
    a = (g * lax.logistic(g)) * u
    out_ref[...] = jnp.dot(a, wd_ref[0], preferred_element_type=jnp.float32)


def _grouped_mlp(xh, xb, W_gate, W_up, W_down, tile_expert):
    boff = D_MODEL // D_BEH
    grid_spec = pltpu.PrefetchScalarGridSpec(
        num_scalar_prefetch=1,
        grid=(NUM_TILES,),
        in_specs=[
            pl.BlockSpec((TILE, D_MODEL), lambda i, te: (i, 0)),
            pl.BlockSpec((TILE, 128), lambda i, te: (i, 0)),
            pl.BlockSpec((1, D_MODEL, D_FF), lambda i, te: (te[i], 0, 0)),
            pl.BlockSpec((1, D_BEH, D_FF), lambda i, te: (te[i], boff, 0)),
            pl.BlockSpec((1, D_MODEL, D_FF), lambda i, te: (te[i], 0, 0)),
            pl.BlockSpec((1, D_BEH, D_FF), lambda i, te: (te[i], boff, 0)),
            pl.BlockSpec((1, D_FF, D_MODEL), lambda i, te: (te[i], 0, 0)),
        ],
        out_specs=pl.BlockSpec((TILE, D_MODEL), lambda i, te: (i, 0)),
    )
    return pl.pallas_call(
        _mlp_body,
        grid_spec=grid_spec,
        out_shape=jax.ShapeDtypeStruct((PAD, D_MODEL), jnp.float32),
        compiler_params=pltpu.CompilerParams(
            vmem_limit_bytes=100 * 1024 * 1024),
    )(tile_expert, xh, xb, W_gate, W_gate, W_up, W_up, W_down)


@jax.jit
def kernel(hidden_states, position_index, behavior_index, behavior_table,
           W_gate, W_up, W_down):
    dest, src_idx, tile_expert = _route(position_index)
    bidx_sorted = behavior_index.astype(jnp.int32)[src_idx]
    bt_pad = jnp.zeros((behavior_table.shape[0], 128), jnp.float32)
    bt_pad = bt_pad.at[:, :D_BEH].set(behavior_table)
    xh, xb = _dispatch_gather(hidden_states, bt_pad, src_idx, bidx_sorted)
    out_sorted = _grouped_mlp(xh, xb, W_gate, W_up, W_down, tile_expert)
    return _combine_gather(out_sorted, dest)


# speedup vs baseline: 2.6092x; 1.0493x over previous
import functools

import jax
import jax.numpy as jnp
from jax import lax
from jax.experimental import pallas as pl
from jax.experimental.pallas import tpu as pltpu
from jax.experimental.pallas import tpu_sc as plsc

NUM_EXPERTS = 8
D_MODEL = 1024
D_BEH = 64
D_FF = 2048
TOKENS = 8192

TILE = 256
NUM_TILES = TOKENS // TILE + NUM_EXPERTS
PAD = NUM_TILES * TILE

NUM_WORKERS = 32
CHUNK = 32


def _route(position_index):
    pi = position_index.astype(jnp.int32)
    onehot = (pi[:, None] == jnp.arange(NUM_EXPERTS, dtype=jnp.int32)[None, :])
    ranks_all = jnp.cumsum(onehot.astype(jnp.int32), axis=0)
    counts = ranks_all[-1]
    rank = jnp.take_along_axis(ranks_all, pi[:, None], axis=1)[:, 0] - 1
    tiles_per_e = (counts + TILE - 1) // TILE
    pad_start = (jnp.concatenate([jnp.zeros((1,), jnp.int32),
                                  jnp.cumsum(tiles_per_e)])[:NUM_EXPERTS]
                 * TILE).astype(jnp.int32)
    dest = pad_start[pi] + rank
    src_idx = jnp.zeros((PAD,), jnp.int32).at[dest].set(
        jnp.arange(TOKENS, dtype=jnp.int32))
    tile_starts = jnp.arange(NUM_TILES, dtype=jnp.int32) * TILE
    tile_expert = (jnp.searchsorted(pad_start, tile_starts, side="right")
                   .astype(jnp.int32) - 1)
    tile_expert = jnp.clip(tile_expert, 0, NUM_EXPERTS - 1)
    return dest, src_idx, tile_expert


def _sc_mesh():
    return plsc.VectorSubcoreMesh(core_axis_name="c", subcore_axis_name="s")


def _dispatch_gather(hidden_states, src_idx):
    rows_per_w = PAD // NUM_WORKERS
    n_chunks = rows_per_w // CHUNK

    @functools.partial(
        pl.kernel,
        out_type=jax.ShapeDtypeStruct((PAD, D_MODEL), jnp.float32),
        mesh=_sc_mesh(),
        scratch_types=[pltpu.VMEM((2, CHUNK), jnp.int32),
                       pltpu.VMEM((2, CHUNK, D_MODEL), jnp.float32),
                       pltpu.SemaphoreType.DMA,
                       pltpu.SemaphoreType.DMA],
    )
    def k(hid_hbm, src_hbm, out_h, idx_v, rows_v, s0, s1):
        wid = lax.axis_index("s") * 2 + lax.axis_index("c")
        base = wid * rows_per_w
        sems = (s0, s1)

        def start(c):
            b = c % 2
            off = base + c * CHUNK
            pltpu.sync_copy(src_hbm.at[pl.ds(off, CHUNK)], idx_v.at[b])
            return pltpu.async_copy(hid_hbm.at[idx_v.at[b]], rows_v.at[b],
                                    sems[b])

        pend = {0: start(0)}
        for c in range(n_chunks):
            if c + 1 < n_chunks:
                pend[c + 1] = start(c + 1)
            pend.pop(c).wait()
            b = c % 2
            off = base + c * CHUNK
            pltpu.sync_copy(rows_v.at[b], out_h.at[pl.ds(off, CHUNK)])

    return k(hidden_states, src_idx)


def _combine_gather(out_sorted, dest):
    rows_per_w = TOKENS // NUM_WORKERS
    n_chunks = rows_per_w // CHUNK

    @functools.partial(
        pl.kernel,
        out_type=jax.ShapeDtypeStruct((TOKENS, D_MODEL), jnp.float32),
        mesh=_sc_mesh(),
        scratch_types=[pltpu.VMEM((2, CHUNK), jnp.int32),
                       pltpu.VMEM((2, CHUNK, D_MODEL), jnp.float32),
                       pltpu.SemaphoreType.DMA,
                       pltpu.SemaphoreType.DMA],
    )
    def k(osort_hbm, dest_hbm, out_hbm, idx_v, rows_v, s0, s1):
        wid = lax.axis_index("s") * 2 + lax.axis_index("c")
        base = wid * rows_per_w
        sems = (s0, s1)

        def start(c):
            b = c % 2
            off = base + c * CHUNK
            pltpu.sync_copy(dest_hbm.at[pl.ds(off, CHUNK)], idx_v.at[b])
            return pltpu.async_copy(osort_hbm.at[idx_v.at[b]], rows_v.at[b],
                                    sems[b])

        pend = {0: start(0)}
        for c in range(n_chunks):
            if c + 1 < n_chunks:
                pend[c + 1] = start(c + 1)
            pend.pop(c).wait()
            b = c % 2
            off = base + c * CHUNK
            pltpu.sync_copy(rows_v.at[b], out_hbm.at[pl.ds(off, CHUNK)])

    return k(out_sorted, dest)


def _mlp_body(te_ref, xh_ref, bidx_ref, bt_ref, wgh_ref, wgb_ref, wuh_ref,
              wub_ref, wd_ref, out_ref):
    xh = xh_ref[...]
    bidx = bidx_ref[0, 0, :]
    onehot = (bidx[:, None]
              == lax.broadcasted_iota(jnp.int32, (TILE, bt_ref.shape[0]), 1))
    xb = jnp.dot(onehot.astype(jnp.float32), bt_ref[...],
                 preferred_element_type=jnp.float32)
    g = (jnp.dot(xh, wgh_ref[0], preferred_element_type=jnp.float32)
         + jnp.dot(xb, wgb_ref[0], preferred_element_type=jnp.float32))
    u = (jnp.dot(xh, wuh_ref[0], preferred_element_type=jnp.float32)
         + jnp.dot(xb, wub_ref[0], preferred_element_type=jnp.float32))
    a = (g * lax.logistic(g)) * u
    out_ref[...] = jnp.dot(a, wd_ref[0], preferred_element_type=jnp.float32)


def _grouped_mlp(xh, bidx_sorted, behavior_table, W_gate, W_up, W_down,
                 tile_expert):
    boff = D_MODEL // D_BEH
    grid_spec = pltpu.PrefetchScalarGridSpec(
        num_scalar_prefetch=1,
        grid=(NUM_TILES,),
        in_specs=[
            pl.BlockSpec((TILE, D_MODEL), lambda i, te: (i, 0)),
            pl.BlockSpec((1, 1, TILE), lambda i, te: (i, 0, 0)),
            pl.BlockSpec(behavior_table.shape, lambda i, te: (0, 0)),
            pl.BlockSpec((1, D_MODEL, D_FF), lambda i, te: (te[i], 0, 0)),
            pl.BlockSpec((1, D_BEH, D_FF), lambda i, te: (te[i], boff, 0)),
            pl.BlockSpec((1, D_MODEL, D_FF), lambda i, te: (te[i], 0, 0)),
            pl.BlockSpec((1, D_BEH, D_FF), lambda i, te: (te[i], boff, 0)),
            pl.BlockSpec((1, D_FF, D_MODEL), lambda i, te: (te[i], 0, 0)),
        ],
        out_specs=pl.BlockSpec((TILE, D_MODEL), lambda i, te: (i, 0)),
    )
    return pl.pallas_call(
        _mlp_body,
        grid_spec=grid_spec,
        out_shape=jax.ShapeDtypeStruct((PAD, D_MODEL), jnp.float32),
        compiler_params=pltpu.CompilerParams(
            vmem_limit_bytes=100 * 1024 * 1024),
    )(tile_expert, xh, bidx_sorted, behavior_table,
      W_gate, W_gate, W_up, W_up, W_down)


@jax.jit
def kernel(hidden_states, position_index, behavior_index, behavior_table,
           W_gate, W_up, W_down):
    dest, src_idx, tile_expert = _route(position_index)
    bidx_sorted = (behavior_index.astype(jnp.int32)[src_idx]
                   .reshape(NUM_TILES, 1, TILE))
    xh = _dispatch_gather(hidden_states, src_idx)
    out_sorted = _grouped_mlp(xh, bidx_sorted, behavior_table,
                              W_gate, W_up, W_down, tile_expert)
    return _combine_gather(out_sorted, dest)


# speedup vs baseline: 3.5586x; 1.3639x over previous
import functools

import jax
import jax.numpy as jnp
from jax import lax
from jax.experimental import pallas as pl
from jax.experimental.pallas import tpu as pltpu
from jax.experimental.pallas import tpu_sc as plsc

NUM_EXPERTS = 8
D_MODEL = 1024
D_BEH = 64
D_FF = 2048
TOKENS = 8192

TILE = 256
NUM_TILES = TOKENS // TILE + NUM_EXPERTS
PAD = NUM_TILES * TILE

NUM_WORKERS = 32
CHUNK = 32


def _route(position_index):
    pi = position_index.astype(jnp.int32)
    onehot = (pi[:, None] == jnp.arange(NUM_EXPERTS, dtype=jnp.int32)[None, :])
    ranks_all = jnp.cumsum(onehot.astype(jnp.int32), axis=0)
    counts = ranks_all[-1]
    rank = jnp.take_along_axis(ranks_all, pi[:, None], axis=1)[:, 0] - 1
    tiles_per_e = (counts + TILE - 1) // TILE
    pad_start = (jnp.concatenate([jnp.zeros((1,), jnp.int32),
                                  jnp.cumsum(tiles_per_e)])[:NUM_EXPERTS]
                 * TILE).astype(jnp.int32)
    dest = pad_start[pi] + rank
    src_idx = (jnp.arange(PAD, dtype=jnp.int32) % TOKENS).at[dest].set(
        jnp.arange(TOKENS, dtype=jnp.int32))
    tile_starts = jnp.arange(NUM_TILES, dtype=jnp.int32) * TILE
    tile_expert = (jnp.searchsorted(pad_start, tile_starts, side="right")
                   .astype(jnp.int32) - 1)
    tile_expert = jnp.clip(tile_expert, 0, NUM_EXPERTS - 1)
    return dest, src_idx, tile_expert


def _sc_mesh():
    return plsc.VectorSubcoreMesh(core_axis_name="c", subcore_axis_name="s")


def _dispatch_gather(hidden_states, src_idx):
    rows_per_w = PAD // NUM_WORKERS
    n_chunks = rows_per_w // CHUNK

    @functools.partial(
        pl.kernel,
        out_type=jax.ShapeDtypeStruct((PAD, D_MODEL), jnp.float32),
        mesh=_sc_mesh(),
        scratch_types=[pltpu.VMEM((2, CHUNK), jnp.int32),
                       pltpu.VMEM((2, CHUNK, D_MODEL), jnp.float32),
                       pltpu.SemaphoreType.DMA,
                       pltpu.SemaphoreType.DMA],
    )
    def k(hid_hbm, src_hbm, out_h, idx_v, rows_v, s0, s1):
        wid = lax.axis_index("s") * 2 + lax.axis_index("c")
        base = wid * rows_per_w
        sems = (s0, s1)

        def start(c):
            b = c % 2
            off = base + c * CHUNK
            pltpu.sync_copy(src_hbm.at[pl.ds(off, CHUNK)], idx_v.at[b])
            return pltpu.async_copy(hid_hbm.at[idx_v.at[b]], rows_v.at[b],
                                    sems[b])

        pend = {0: start(0)}
        for c in range(n_chunks):
            if c + 1 < n_chunks:
                pend[c + 1] = start(c + 1)
            pend.pop(c).wait()
            b = c % 2
            off = base + c * CHUNK
            pltpu.sync_copy(rows_v.at[b], out_h.at[pl.ds(off, CHUNK)])

    return k(hidden_states, src_idx)


def _combine_gather(out_sorted, dest):
    rows_per_w = TOKENS // NUM_WORKERS
    n_chunks = rows_per_w // CHUNK

    @functools.partial(
        pl.kernel,
        out_type=jax.ShapeDtypeStruct((TOKENS, D_MODEL), jnp.float32),
        mesh=_sc_mesh(),
        scratch_types=[pltpu.VMEM((2, CHUNK), jnp.int32),
                       pltpu.VMEM((2, CHUNK, D_MODEL), jnp.float32),
                       pltpu.SemaphoreType.DMA,
                       pltpu.SemaphoreType.DMA],
    )
    def k(osort_hbm, dest_hbm, out_hbm, idx_v, rows_v, s0, s1):
        wid = lax.axis_index("s") * 2 + lax.axis_index("c")
        base = wid * rows_per_w
        sems = (s0, s1)

        def start(c):
            b = c % 2
            off = base + c * CHUNK
            pltpu.sync_copy(dest_hbm.at[pl.ds(off, CHUNK)], idx_v.at[b])
            return pltpu.async_copy(osort_hbm.at[idx_v.at[b]], rows_v.at[b],
                                    sems[b])

        pend = {0: start(0)}
        for c in range(n_chunks):
            if c + 1 < n_chunks:
                pend[c + 1] = start(c + 1)
            pend.pop(c).wait()
            b = c % 2
            off = base + c * CHUNK
            pltpu.sync_copy(rows_v.at[b], out_hbm.at[pl.ds(off, CHUNK)])

    return k(out_sorted, dest)


def _mlp_body(te_ref, xh_ref, bidx_ref, bt_ref, wgh_ref, wgb_ref, wuh_ref,
              wub_ref, wd_ref, out_ref):
    xh = xh_ref[...]
    bidx = bidx_ref[0, 0, :]
    onehot = (bidx[:, None]
              == lax.broadcasted_iota(jnp.int32, (TILE, bt_ref.shape[0]), 1))
    xb = jnp.dot(onehot.astype(jnp.float32), bt_ref[...],
                 preferred_element_type=jnp.float32)
    g = (jnp.dot(xh, wgh_ref[0], preferred_element_type=jnp.float32)
         + jnp.dot(xb, wgb_ref[0], preferred_element_type=jnp.float32))
    u = (jnp.dot(xh, wuh_ref[0], preferred_element_type=jnp.float32)
         + jnp.dot(xb, wub_ref[0], preferred_element_type=jnp.float32))
    a = (g * lax.logistic(g)) * u
    out_ref[...] = jnp.dot(a, wd_ref[0], preferred_element_type=jnp.float32)


def _grouped_mlp(xh, bidx_sorted, behavior_table, W_gate, W_up, W_down,
                 tile_expert):
    boff = D_MODEL // D_BEH
    grid_spec = pltpu.PrefetchScalarGridSpec(
        num_scalar_prefetch=1,
        grid=(NUM_TILES,),
        in_specs=[
            pl.BlockSpec((TILE, D_MODEL), lambda i, te: (i, 0)),
            pl.BlockSpec((1, 1, TILE), lambda i, te: (i, 0, 0)),
            pl.BlockSpec(behavior_table.shape, lambda i, te: (0, 0)),
            pl.BlockSpec((1, D_MODEL, D_FF), lambda i, te: (te[i], 0, 0)),
            pl.BlockSpec((1, D_BEH, D_FF), lambda i, te: (te[i], boff, 0)),
            pl.BlockSpec((1, D_MODEL, D_FF), lambda i, te: (te[i], 0, 0)),
            pl.BlockSpec((1, D_BEH, D_FF), lambda i, te: (te[i], boff, 0)),
            pl.BlockSpec((1, D_FF, D_MODEL), lambda i, te: (te[i], 0, 0)),
        ],
        out_specs=pl.BlockSpec((TILE, D_MODEL), lambda i, te: (i, 0)),
    )
    return pl.pallas_call(
        _mlp_body,
        grid_spec=grid_spec,
        out_shape=jax.ShapeDtypeStruct((PAD, D_MODEL), jnp.float32),
        compiler_params=pltpu.CompilerParams(
            vmem_limit_bytes=100 * 1024 * 1024),
    )(tile_expert, xh, bidx_sorted, behavior_table,
      W_gate, W_gate, W_up, W_up, W_down)


@jax.jit
def kernel(hidden_states, position_index, behavior_index, behavior_table,
           W_gate, W_up, W_down):
    dest, src_idx, tile_expert = _route(position_index)
    bidx_sorted = (behavior_index.astype(jnp.int32)[src_idx]
                   .reshape(NUM_TILES, 1, TILE))
    xh = _dispatch_gather(hidden_states, src_idx)
    out_sorted = _grouped_mlp(xh, bidx_sorted, behavior_table,
                              W_gate, W_up, W_down, tile_expert)
    return _combine_gather(out_sorted, dest)
